# Initial kernel scaffold; baseline (speedup 1.0000x reference)
import jax
import jax.numpy as jnp
from jax.experimental import pallas as pl


def kernel(x, edge_index, batch, W1, b1, W2, b2, W3, b3, Wc1, bc1, Wc2, bc2, Wfc, bfc):
    raise NotImplementedError("write your pallas kernel here")


# speedup vs baseline: 12.7109x; 12.7109x over previous
import functools

import jax
import jax.numpy as jnp
from jax import lax
from jax.experimental import pallas as pl
from jax.experimental.pallas import tpu as pltpu
from jax.experimental.pallas import tpu_sc as plsc

N = 10000
NP = 10240
E = 320000
F_IN = 128
H = 64
B = 64
K = 10

NCORES = 2
NSUB = 16
NW = NCORES * NSUB
CHUNK = 128
CH = 80
EP = NW * CH * CHUNK
RPT = NP // NSUB

_mesh = plsc.VectorSubcoreMesh(core_axis_name="c", subcore_axis_name="s")


@functools.partial(
    pl.kernel,
    out_type=jax.ShapeDtypeStruct((NCORES * NP,), jnp.float32),
    mesh=_mesh,
    scratch_types=[
        pltpu.VMEM((CH, CHUNK), jnp.int32),
        pltpu.VMEM((CHUNK,), jnp.float32),
        pltpu.VMEM_SHARED((NP,), jnp.float32),
    ],
    compiler_params=pltpu.CompilerParams(use_tc_tiling_on_sc=False),
)
def _sc_degree(dst_hbm, z_hbm, out_hbm, didx, ones, acc):
    c = lax.axis_index("c")
    s = lax.axis_index("s")
    wid = s * NCORES + c
    for i in range(CHUNK // 16):
        ones[pl.ds(i * 16, 16)] = jnp.ones((16,), jnp.float32)
    sl = pl.ds(s * RPT, RPT)
    pltpu.sync_copy(z_hbm.at[sl], acc.at[sl])
    pltpu.sync_copy(dst_hbm.at[pl.ds(wid * CH, CH)], didx)
    plsc.subcore_barrier()

    def body(j, carry):
        pltpu.sync_copy(ones, acc.at[didx.at[j]], add=True)
        return carry

    lax.fori_loop(0, CH, body, 0)
    plsc.subcore_barrier()
    pltpu.sync_copy(acc.at[sl], out_hbm.at[pl.ds(c * NP + s * RPT, RPT)])


@functools.partial(
    pl.kernel,
    out_type=jax.ShapeDtypeStruct((NCORES, NP, H), jnp.float32),
    mesh=_mesh,
    scratch_types=[
        pltpu.VMEM((CH, CHUNK), jnp.int32),
        pltpu.VMEM((CH, CHUNK), jnp.int32),
        pltpu.VMEM((CHUNK, H), jnp.float32),
        pltpu.VMEM_SHARED((NP, H), jnp.float32),
        pltpu.SemaphoreType.DMA,
    ],
    compiler_params=pltpu.CompilerParams(use_tc_tiling_on_sc=False),
)
def _sc_scatter(m_hbm, src_hbm, dst_hbm, z_hbm, out_hbm,
                sidx, didx, rows, acc, sem):
    c = lax.axis_index("c")
    s = lax.axis_index("s")
    wid = s * NCORES + c
    pltpu.sync_copy(z_hbm.at[pl.ds(s * RPT, RPT)], acc.at[pl.ds(s * RPT, RPT)])
    pltpu.sync_copy(src_hbm.at[pl.ds(wid * CH, CH)], sidx)
    pltpu.sync_copy(dst_hbm.at[pl.ds(wid * CH, CH)], didx)
    plsc.subcore_barrier()

    def body(j, carry):
        pltpu.async_copy(m_hbm.at[sidx.at[j]], rows, sem).wait()
        pltpu.sync_copy(rows, acc.at[didx.at[j]], add=True)
        return carry

    lax.fori_loop(0, CH, body, 0)
    plsc.subcore_barrier()
    pltpu.sync_copy(acc.at[pl.ds(s * RPT, RPT)],
                    out_hbm.at[c, pl.ds(s * RPT, RPT)])


def _dinv_body(d0_ref, d1_ref, out_ref):
    d = d0_ref[...] + d1_ref[...] + 1.0
    out_ref[...] = lax.rsqrt(d)


def _tc_dinv(degp):
    deg2 = degp.reshape(NCORES, NP // 128, 128)
    return pl.pallas_call(
        _dinv_body,
        out_shape=jax.ShapeDtypeStruct((NP // 128, 128), jnp.float32),
    )(deg2[0], deg2[1])


def _m1_body(x_ref, w_ref, dinv_ref, out_ref):
    mm = jnp.dot(x_ref[...], w_ref[...], preferred_element_type=jnp.float32)
    out_ref[...] = mm * dinv_ref[...]


def _tc_m1(xp, W1, dinv):
    blk = 1280
    g = NP // blk
    return pl.pallas_call(
        _m1_body,
        grid=(g,),
        in_specs=[
            pl.BlockSpec((blk, F_IN), lambda i: (i, 0)),
            pl.BlockSpec((F_IN, H), lambda i: (0, 0)),
            pl.BlockSpec((blk, 1), lambda i: (i, 0)),
        ],
        out_specs=pl.BlockSpec((blk, H), lambda i: (i, 0)),
        out_shape=jax.ShapeDtypeStruct((NP, H), jnp.float32),
    )(xp, W1, dinv)


def _layer_body(sp_ref, m_ref, dinv_ref, b_ref, w_ref, xprev_ref, mnext_ref):
    t = (sp_ref[0] + sp_ref[1] + m_ref[...]) * dinv_ref[...] + b_ref[...]
    xprev = jnp.tanh(t)
    xprev_ref[...] = xprev
    mm = jnp.dot(xprev, w_ref[...], preferred_element_type=jnp.float32)
    mnext_ref[...] = mm * dinv_ref[...]


def _tc_layer(sp, m, dinv, b, W):
    blk = 1280
    g = NP // blk
    return pl.pallas_call(
        _layer_body,
        grid=(g,),
        in_specs=[
            pl.BlockSpec((NCORES, blk, H), lambda i: (0, i, 0)),
            pl.BlockSpec((blk, H), lambda i: (i, 0)),
            pl.BlockSpec((blk, 1), lambda i: (i, 0)),
            pl.BlockSpec((1, H), lambda i: (0, 0)),
            pl.BlockSpec((H, H), lambda i: (0, 0)),
        ],
        out_specs=[
            pl.BlockSpec((blk, H), lambda i: (i, 0)),
            pl.BlockSpec((blk, H), lambda i: (i, 0)),
        ],
        out_shape=[
            jax.ShapeDtypeStruct((NP, H), jnp.float32),
            jax.ShapeDtypeStruct((NP, H), jnp.float32),
        ],
    )(sp, m, dinv, b, W)


def _final_body(sp_ref, m_ref, dinv_ref, b_ref, x3_ref):
    t = (sp_ref[0] + sp_ref[1] + m_ref[...]) * dinv_ref[...] + b_ref[...]
    x3_ref[...] = jnp.tanh(t)


def _tc_final(sp, m, dinv, b):
    blk = 1280
    g = NP // blk
    return pl.pallas_call(
        _final_body,
        grid=(g,),
        in_specs=[
            pl.BlockSpec((NCORES, blk, H), lambda i: (0, i, 0)),
            pl.BlockSpec((blk, H), lambda i: (i, 0)),
            pl.BlockSpec((blk, 1), lambda i: (i, 0)),
            pl.BlockSpec((1, H), lambda i: (0, 0)),
        ],
        out_specs=pl.BlockSpec((blk, H), lambda i: (i, 0)),
        out_shape=jax.ShapeDtypeStruct((NP, H), jnp.float32),
    )(sp, m, dinv, b)


def kernel(x, edge_index, batch, W1, b1, W2, b2, W3, b3,
           Wc1, bc1, Wc2, bc2, Wfc, bfc):
    src = edge_index[0].astype(jnp.int32)
    dst = edge_index[1].astype(jnp.int32)
    pad = jnp.full((EP - E,), N, dtype=jnp.int32)
    src2 = jnp.concatenate([src, pad]).reshape(NW * CH, CHUNK)
    dst2 = jnp.concatenate([dst, pad]).reshape(NW * CH, CHUNK)

    xp = jnp.zeros((NP, F_IN), jnp.float32).at[:N].set(x)
    z1 = jnp.zeros((NP,), jnp.float32)
    z2 = jnp.zeros((NP, H), jnp.float32)

    degp = _sc_degree(dst2, z1)
    dinv = _tc_dinv(degp).reshape(NP, 1)

    m1 = _tc_m1(xp, W1, dinv)
    s1 = _sc_scatter(m1, src2, dst2, z2)
    x1, m2 = _tc_layer(s1, m1, dinv, b1.reshape(1, H), W2)
    s2 = _sc_scatter(m2, src2, dst2, z2)
    x2, m3 = _tc_layer(s2, m2, dinv, b2.reshape(1, H), W3)
    s3 = _sc_scatter(m3, src2, dst2, z2)
    x3 = _tc_final(s3, m3, dinv, b3.reshape(1, H))

    xc = jnp.concatenate([x1[:N], x2[:N], x3[:N]], axis=1)
    keys = xc[:, -1]
    NEG = jnp.float32(-1e30)
    mask = batch[None, :] == jnp.arange(B)[:, None]
    masked = jnp.where(mask, keys[None, :], NEG)
    topv, topi = jax.lax.top_k(masked, K)
    feats = xc[topi]
    feats = jnp.where((topv > NEG / 2)[..., None], feats, 0.0)
    dn = ("NCH", "OIH", "NCH")
    y = jax.lax.conv_general_dilated(feats, Wc1, (4,), "VALID",
                                     dimension_numbers=dn) + bc1[None, :, None]
    y = jax.nn.relu(y)
    b_, c_, l_ = y.shape
    y = y[:, :, : (l_ // 4) * 4].reshape(b_, c_, l_ // 4, 4).max(-1)
    y = jax.lax.conv_general_dilated(y, Wc2, (3,), "VALID",
                                     dimension_numbers=dn) + bc2[None, :, None]
    y = jax.nn.relu(y)
    b_, c_, l_ = y.shape
    y = y[:, :, : (l_ // 4) * 4].reshape(b_, c_, l_ // 4, 4).max(-1)
    xflat = y.reshape(b_, -1)
    c = jax.nn.relu(xflat) @ Wfc + bfc
    return (c, xflat)
